# baseline (speedup 1.0000x reference)
import functools

import jax
import jax.numpy as jnp
from jax import lax
from jax.experimental import pallas as pl
from jax.experimental.pallas import tpu as pltpu
from jax.experimental.pallas import tpu_sc as plsc

D = 17
D2 = D * D
D3 = D2 * D
N_LATTICE = D ** 4
B, H, W = 2, 512, 512
HW = H * W
NPIX = B * HW
NSC = 2
NTILES = 32
PIX_PER_TILE = NPIX // NTILES
PCH = 256
NGRP = PCH // 16
NCHUNK = PIX_PER_TILE // PCH
NROW = 4 * PCH
GOFF = (0, D2, D3, D3 + D2)
SHIFTS = (0, 1, D, D + 1)

NPAD = 98304
RPT = NPAD // 16
SEG = RPT + 32
SRC_PAD = NPAD + 32
BCH = 1024
NBCH = RPT // BCH
NTAB = 6


def _sc_kernel():
    mesh = plsc.VectorSubcoreMesh(core_axis_name="c", subcore_axis_name="s")

    @functools.partial(
        pl.kernel,
        mesh=mesh,
        compiler_params=pltpu.CompilerParams(
            needs_layout_passes=False, use_tc_tiling_on_sc=False),
        out_type=(
            jax.ShapeDtypeStruct((B * 3 * HW,), jnp.float32),
            jax.ShapeDtypeStruct((NSC * NTAB * NPAD, 16), jnp.float32),
        ),
        scratch_types=[
            pltpu.VMEM((4, PIX_PER_TILE), jnp.float32),
            pltpu.VMEM((4, SEG), jnp.float32),
            pltpu.VMEM((4, PCH), jnp.float32),
            pltpu.VMEM((4, PCH), jnp.float32),
            pltpu.VMEM((NROW,), jnp.int32),
            pltpu.VMEM((NROW,), jnp.int32),
            pltpu.VMEM((NROW, 16), jnp.float32),
            pltpu.VMEM((NROW, 16), jnp.float32),
            pltpu.SemaphoreType.DMA,
            pltpu.SemaphoreType.DMA,
        ],
    )
    def k(con, luts, out, tab, xbuf, srcbuf, fbufa, fbufb, idxa, idxb,
          rowsa, rowsb, sema, semb):
        cid = lax.axis_index("c")
        sid = lax.axis_index("s")
        wid = sid * NSC + cid
        base_pix = wid * PIX_PER_TILE
        in_extra = jnp.where(base_pix >= HW, 3 * HW, 0)
        out_extra = jnp.where(base_pix >= HW, 2 * HW, 0)
        for ch in range(4):
            pltpu.sync_copy(
                con.at[pl.ds(base_pix + in_extra + ch * HW, PIX_PER_TILE)],
                xbuf.at[ch])
        lanes = lax.iota(jnp.int32, 16)
        cols = [jnp.full((16,), v, jnp.int32) for v in range(16)]

        sc_tab0 = cid * (NTAB * NPAD)
        r0 = sid * RPT

        def build_table(s, _):
            for ch in range(4):
                pltpu.sync_copy(luts.at[s, ch, pl.ds(r0, SEG)], srcbuf.at[ch])

            def build_chunk(chunk, _):
                @plsc.parallel_loop(0, BCH // 16)
                def _(g):
                    rl = lanes + g * 16
                    for kl, sh in enumerate(SHIFTS):
                        for ch in range(4):
                            v = srcbuf[ch, pl.ds(chunk * BCH + g * 16 + sh, 16)]
                            plsc.store_scatter(rowsa, [rl, cols[kl * 4 + ch]], v)

                pltpu.sync_copy(
                    rowsa.at[pl.ds(0, BCH)],
                    tab.at[pl.ds(sc_tab0 + s * NPAD + r0 + chunk * BCH, BCH)])
                return 0

            lax.fori_loop(0, NBCH, build_chunk, 0)
            return 0

        lax.fori_loop(0, NTAB, build_table, 0)
        plsc.subcore_barrier()

        def stage(sbase, n_out, do_clip):
            def p1(ci, fbuf, idx):
                coff = ci * PCH

                @plsc.parallel_loop(0, NGRP)
                def _(g):
                    off = coff + g * 16
                    q = []
                    for ch in range(4):
                        x = xbuf[ch, pl.ds(off, 16)]
                        v = jnp.minimum(jnp.maximum(x, 0.0), 1.0) * float(D - 1)
                        qi = jnp.minimum(v.astype(jnp.int32), D - 2)
                        fbuf[ch, pl.ds(g * 16, 16)] = v - qi.astype(jnp.float32)
                        q.append(qi)
                    n0 = ((q[0] * D + q[1]) * D + q[2]) * D + q[3] + sbase
                    for gi in range(4):
                        idx[pl.ds(gi * PCH + g * 16, 16)] = n0 + GOFF[gi]

            def fire(idx, rows, sem):
                for j in range(NROW // 128):
                    pltpu.async_copy(
                        tab.at[idx.at[pl.ds(j * 128, 128)]],
                        rows.at[pl.ds(j * 128, 128)], sem)

            def drain(rows, sem):
                for j in range(NROW // 128):
                    pltpu.make_async_copy(
                        tab.at[pl.ds(0, 128)], rows.at[pl.ds(j * 128, 128)],
                        sem).wait()

            def p2(ci, fbuf, rows):
                coff = ci * PCH

                @plsc.parallel_loop(0, NGRP)
                def _(g):
                    off = coff + g * 16
                    fi = fbuf[0, pl.ds(g * 16, 16)]
                    fj = fbuf[1, pl.ds(g * 16, 16)]
                    fk = fbuf[2, pl.ds(g * 16, 16)]
                    fl = fbuf[3, pl.ds(g * 16, 16)]
                    gi0, gj0 = 1.0 - fi, 1.0 - fj
                    gk0, gl0 = 1.0 - fk, 1.0 - fl
                    wg = (gi0 * gj0, gi0 * fj, fi * gj0, fi * fj)
                    wkl = (gk0 * gl0, gk0 * fl, fk * gl0, fk * fl)
                    acc = [None] * n_out
                    for gi in range(4):
                        ridx = lanes + (gi * PCH + g * 16)
                        for ch in range(n_out):
                            s = wkl[0] * plsc.load_gather(rows, [ridx, cols[ch]])
                            for kl in range(1, 4):
                                s = s + wkl[kl] * plsc.load_gather(
                                    rows, [ridx, cols[kl * 4 + ch]])
                            t = wg[gi] * s
                            acc[ch] = t if acc[ch] is None else acc[ch] + t
                    for ch in range(n_out):
                        v = acc[ch]
                        if do_clip:
                            v = jnp.minimum(jnp.maximum(v, 0.0), 1.0)
                        xbuf[ch, pl.ds(off, 16)] = v

            p1(0, fbufa, idxa)
            fire(idxa, rowsa, sema)

            def pair_body(kk, _):
                c0 = 2 * kk
                p1(c0 + 1, fbufb, idxb)
                fire(idxb, rowsb, semb)
                drain(rowsa, sema)
                p2(c0, fbufa, rowsa)

                @pl.when(kk < NCHUNK // 2 - 1)
                def _():
                    p1(c0 + 2, fbufa, idxa)
                    fire(idxa, rowsa, sema)

                drain(rowsb, semb)
                p2(c0 + 1, fbufb, rowsb)
                return 0

            lax.fori_loop(0, NCHUNK // 2, pair_body, 0)

        def stage_body(s, _):
            stage(sc_tab0 + s * NPAD, 4, True)
            return 0

        lax.fori_loop(0, NTAB - 1, stage_body, 0)
        stage(sc_tab0 + (NTAB - 1) * NPAD, 3, False)

        for ch in range(3):
            pltpu.sync_copy(
                xbuf.at[ch],
                out.at[pl.ds(base_pix + out_extra + ch * HW, PIX_PER_TILE)])

    return k


_K = None


def kernel(vi_image, ir_image, LUT8, LUT00, LUT01, LUT02, LUT03, LUTPGF):
    global _K
    if _K is None:
        _K = _sc_kernel()
    con = jnp.concatenate([vi_image, ir_image], axis=1).reshape(-1)
    lfs = []
    for lut in (LUT8, LUT00, LUT01, LUT02, LUT03, LUTPGF):
        lf = lut.reshape(lut.shape[0], -1)
        if lf.shape[0] < 4:
            lf = jnp.concatenate(
                [lf, jnp.zeros((4 - lf.shape[0], lf.shape[1]), lf.dtype)], 0)
        lfs.append(jnp.pad(lf, ((0, 0), (0, SRC_PAD - N_LATTICE))))
    luts = jnp.stack(lfs, 0)
    out, _ = _K(con, luts)
    return out.reshape(B, 3, H, W)
